# baseline (speedup 1.0000x reference)
import functools

import jax
import jax.numpy as jnp
from jax import lax
from jax.experimental import pallas as pl
from jax.experimental.pallas import tpu as pltpu
from jax.experimental.pallas import tpu_sc as plsc

N = 102400
E = 409600
C = 128
G = 4
CG = C // G
NSC = 2
NT = 16
EPT = E // (NSC * NT)
EPS = E // NT
ECH = 3200
SB = 400
STRIPE = N // NT
ROWBLK = 2048
GRID = N // ROWBLK

_mesh = plsc.VectorSubcoreMesh(core_axis_name="c", subcore_axis_name="s")
_sc_params = pltpu.CompilerParams(use_tc_tiling_on_sc=False)


@functools.partial(
    pl.kernel,
    mesh=_mesh,
    out_type=jax.ShapeDtypeStruct((NSC, N), jnp.float32),
    compiler_params=_sc_params,
    scratch_types=[
        pltpu.VMEM((EPT,), jnp.int32),
        pltpu.VMEM((1, 128), jnp.int32),
        pltpu.VMEM((128,), jnp.float32),
        pltpu.VMEM((STRIPE,), jnp.float32),
        pltpu.VMEM_SHARED((N,), jnp.float32),
    ],
)
def _degree_kernel(col_hbm, cnt_hbm, colchunk, colstage, ones_v, stage1d, spdeg):
    cid = lax.axis_index("c")
    sid = lax.axis_index("s")
    wid = sid * NSC + cid

    def _z(j, carry):
        stage1d[pl.ds(j * 16, 16)] = jnp.zeros((16,), jnp.float32)
        return carry

    lax.fori_loop(0, STRIPE // 16, _z, 0)
    for k in range(8):
        ones_v[pl.ds(k * 16, 16)] = jnp.ones((16,), jnp.float32)

    pltpu.sync_copy(stage1d, spdeg.at[pl.ds(sid * STRIPE, STRIPE)])
    plsc.subcore_barrier()

    pltpu.sync_copy(col_hbm.at[pl.ds(wid * EPT, EPT)], colchunk)

    def _hist(b, carry):
        for k in range(8):
            colstage[0, pl.ds(k * 16, 16)] = colchunk[pl.ds(b * 128 + k * 16, 16)]
        pltpu.sync_copy(ones_v, spdeg.at[colstage.at[0]], add=True)
        return carry

    lax.fori_loop(0, EPT // 128, _hist, 0)
    plsc.subcore_barrier()

    pltpu.sync_copy(spdeg.at[pl.ds(sid * STRIPE, STRIPE)], stage1d)
    pltpu.sync_copy(stage1d, cnt_hbm.at[cid, pl.ds(sid * STRIPE, STRIPE)])


@functools.partial(
    pl.kernel,
    mesh=_mesh,
    out_type=jax.ShapeDtypeStruct((N, C), jnp.bfloat16),
    compiler_params=_sc_params,
    scratch_types=[
        pltpu.VMEM((ECH,), jnp.int32),
        pltpu.VMEM((ECH,), jnp.int32),
        pltpu.VMEM((SB,), jnp.int32),
        pltpu.VMEM((SB,), jnp.int32),
        pltpu.VMEM((SB,), jnp.int32),
        pltpu.VMEM((SB,), jnp.int32),
        pltpu.VMEM((2, SB, CG), jnp.bfloat16),
        pltpu.VMEM_SHARED((N, CG), jnp.bfloat16),
        pltpu.SemaphoreType.DMA,
        pltpu.SemaphoreType.DMA,
    ],
)
def _scatter_kernel(hd_hbm, row_hbm, col_hbm, acc_hbm,
                    rowchunk, colchunk, rowidx0, colidx0, rowidx1, colidx1,
                    stage, spacc, sem_g, sem_s):
    cid = lax.axis_index("c")
    sid = lax.axis_index("s")
    rixs = (rowidx0, rowidx1)
    cixs = (colidx0, colidx1)

    for p in range(G // NSC):
        g = p * NSC + cid

        def _zz(j, carry):
            stage[0, j, :] = jnp.zeros((CG,), jnp.bfloat16)
            return carry

        lax.fori_loop(0, SB, _zz, 0)
        zh = [
            pltpu.async_copy(
                stage.at[0], spacc.at[pl.ds(sid * STRIPE + k * SB, SB)],
                sem_g)
            for k in range(STRIPE // SB)
        ]
        for h in zh:
            h.wait()
        plsc.subcore_barrier()

        def _chunk(cki, carry):
            e0 = sid * EPS + cki * ECH
            pltpu.sync_copy(row_hbm.at[pl.ds(e0, ECH)], rowchunk)
            pltpu.sync_copy(col_hbm.at[pl.ds(e0, ECH)], colchunk)

            gh, sh = {}, {}
            for k in range(ECH // SB):
                kb = k % 2
                if k >= 2:
                    sh[k - 2].wait()

                def _bld(j, carry2, _k=k, _kb=kb):
                    rv = rowchunk[pl.ds(_k * SB + j * 16, 16)]
                    cv = colchunk[pl.ds(_k * SB + j * 16, 16)]
                    rixs[_kb][pl.ds(j * 16, 16)] = rv * G + g
                    cixs[_kb][pl.ds(j * 16, 16)] = cv
                    return carry2

                lax.fori_loop(0, SB // 16, _bld, 0)
                gh[k] = pltpu.async_copy(
                    hd_hbm.at[rixs[kb]], stage.at[kb], sem_g)
                if k >= 1:
                    gh[k - 1].wait()
                    sh[k - 1] = pltpu.async_copy(
                        stage.at[(k - 1) % 2], spacc.at[cixs[(k - 1) % 2]],
                        sem_s, add=True)
            last = ECH // SB - 1
            gh[last].wait()
            sh[last] = pltpu.async_copy(
                stage.at[last % 2], spacc.at[cixs[last % 2]], sem_s,
                add=True)
            sh[last - 1].wait()
            sh[last].wait()
            return carry

        lax.fori_loop(0, EPS // ECH, _chunk, 0)
        plsc.subcore_barrier()

        dh = {}
        for k in range(STRIPE // SB):
            kb = k % 2
            if k >= 2:
                dh[k - 2].wait()
            r0 = sid * STRIPE + k * SB
            pltpu.sync_copy(spacc.at[pl.ds(r0, SB)], stage.at[kb])
            dh[k] = pltpu.async_copy(
                stage.at[kb],
                acc_hbm.at[pl.ds(r0, SB), pl.ds(g * CG, CG)], sem_s)
        dh[STRIPE // SB - 2].wait()
        dh[STRIPE // SB - 1].wait()


def _hd_body(x_ref, c0_ref, c1_ref, w_ref, hd_ref, dis_ref):
    deg = c0_ref[...] + c1_ref[...] + 1.0
    dis = lax.rsqrt(deg)
    h = jnp.dot(x_ref[...], w_ref[...], preferred_element_type=jnp.float32)
    hd_ref[...] = (h * dis).astype(jnp.bfloat16)
    dis_ref[...] = dis


_hd_call = pl.pallas_call(
    _hd_body,
    grid=(GRID,),
    in_specs=[
        pl.BlockSpec((ROWBLK, C), lambda i: (i, 0)),
        pl.BlockSpec((ROWBLK, 1), lambda i: (i, 0)),
        pl.BlockSpec((ROWBLK, 1), lambda i: (i, 0)),
        pl.BlockSpec((C, C), lambda i: (0, 0)),
    ],
    out_specs=[
        pl.BlockSpec((ROWBLK, C), lambda i: (i, 0)),
        pl.BlockSpec((ROWBLK, 1), lambda i: (i, 0)),
    ],
    out_shape=[
        jax.ShapeDtypeStruct((N, C), jnp.bfloat16),
        jax.ShapeDtypeStruct((N, 1), jnp.float32),
    ],
)


def _y_body(acc_ref, hd_ref, dis_ref, b_ref, y_ref, sums_ref):
    i = pl.program_id(0)
    acc = acc_ref[...].astype(jnp.float32)
    hd = hd_ref[...].astype(jnp.float32)
    y = (acc + hd) * dis_ref[...] + b_ref[...]
    y_ref[...] = y
    s1 = jnp.sum(y, axis=0, keepdims=True)
    s2 = jnp.sum(y * y, axis=0, keepdims=True)
    blk = jnp.concatenate([s1, s2, jnp.zeros((6, C), jnp.float32)], axis=0)

    @pl.when(i == 0)
    def _():
        sums_ref[...] = blk

    @pl.when(i > 0)
    def _():
        sums_ref[...] = sums_ref[...] + blk


_y_call = pl.pallas_call(
    _y_body,
    grid=(GRID,),
    in_specs=[
        pl.BlockSpec((ROWBLK, C), lambda i: (i, 0)),
        pl.BlockSpec((ROWBLK, C), lambda i: (i, 0)),
        pl.BlockSpec((ROWBLK, 1), lambda i: (i, 0)),
        pl.BlockSpec((1, C), lambda i: (0, 0)),
    ],
    out_specs=[
        pl.BlockSpec((ROWBLK, C), lambda i: (i, 0)),
        pl.BlockSpec((8, C), lambda i: (0, 0)),
    ],
    out_shape=[
        jax.ShapeDtypeStruct((N, C), jnp.float32),
        jax.ShapeDtypeStruct((8, C), jnp.float32),
    ],
)


def _bn_body(y_ref, a_ref, c_ref, o_ref):
    z = y_ref[...] * a_ref[...] + c_ref[...]
    o_ref[...] = jnp.where(z >= 0, z, 0.2 * z)


_bn_call = pl.pallas_call(
    _bn_body,
    grid=(GRID,),
    in_specs=[
        pl.BlockSpec((ROWBLK, C), lambda i: (i, 0)),
        pl.BlockSpec((1, C), lambda i: (0, 0)),
        pl.BlockSpec((1, C), lambda i: (0, 0)),
    ],
    out_specs=pl.BlockSpec((ROWBLK, C), lambda i: (i, 0)),
    out_shape=jax.ShapeDtypeStruct((N, C), jnp.float32),
)


def kernel(x, edge_index, batch_size, num_frames, num_joints, W, b, gamma, beta):
    row = edge_index[0]
    col = edge_index[1]

    cnt = _degree_kernel(col)
    c0 = cnt[0].reshape(N, 1)
    c1 = cnt[1].reshape(N, 1)

    hd, dis = _hd_call(x, c0, c1, W)

    acc = _scatter_kernel(hd.reshape(N * G, CG), row, col)

    y, sums = _y_call(acc, hd, dis, b.reshape(1, C))
    mean = sums[0] / N
    var = sums[1] / N - mean * mean
    inv = lax.rsqrt(var + 1e-5)
    a = inv * gamma
    c = beta - mean * a

    out = _bn_call(y, a.reshape(1, C), c.reshape(1, C))
    return out.reshape(64, 64, 25, C)


# speedup vs baseline: 1.1241x; 1.1241x over previous
import functools

import jax
import jax.numpy as jnp
from jax import lax
from jax.experimental import pallas as pl
from jax.experimental.pallas import tpu as pltpu
from jax.experimental.pallas import tpu_sc as plsc

N = 102400
E = 409600
C = 128
G = 4
CG = C // G
NSC = 2
NT = 16
EPT = E // (NSC * NT)
EPS = E // NT
ECH = 3200
SB = 400
STRIPE = N // NT
ROWBLK = 2048
GRID = N // ROWBLK

_mesh = plsc.VectorSubcoreMesh(core_axis_name="c", subcore_axis_name="s")
_sc_params = pltpu.CompilerParams(use_tc_tiling_on_sc=False)


@functools.partial(
    pl.kernel,
    mesh=_mesh,
    out_type=jax.ShapeDtypeStruct((NSC, N), jnp.float32),
    compiler_params=_sc_params,
    scratch_types=[
        pltpu.VMEM((EPT,), jnp.int32),
        pltpu.VMEM((1, 128), jnp.int32),
        pltpu.VMEM((128,), jnp.float32),
        pltpu.VMEM((STRIPE,), jnp.float32),
        pltpu.VMEM_SHARED((N,), jnp.float32),
    ],
)
def _degree_kernel(col_hbm, cnt_hbm, colchunk, colstage, ones_v, stage1d, spdeg):
    cid = lax.axis_index("c")
    sid = lax.axis_index("s")
    wid = sid * NSC + cid

    def _z(j, carry):
        stage1d[pl.ds(j * 16, 16)] = jnp.zeros((16,), jnp.float32)
        return carry

    lax.fori_loop(0, STRIPE // 16, _z, 0)
    for k in range(8):
        ones_v[pl.ds(k * 16, 16)] = jnp.ones((16,), jnp.float32)

    pltpu.sync_copy(stage1d, spdeg.at[pl.ds(sid * STRIPE, STRIPE)])
    plsc.subcore_barrier()

    pltpu.sync_copy(col_hbm.at[pl.ds(wid * EPT, EPT)], colchunk)

    def _hist(b, carry):
        for k in range(8):
            colstage[0, pl.ds(k * 16, 16)] = colchunk[pl.ds(b * 128 + k * 16, 16)]
        pltpu.sync_copy(ones_v, spdeg.at[colstage.at[0]], add=True)
        return carry

    lax.fori_loop(0, EPT // 128, _hist, 0)
    plsc.subcore_barrier()

    pltpu.sync_copy(spdeg.at[pl.ds(sid * STRIPE, STRIPE)], stage1d)
    pltpu.sync_copy(stage1d, cnt_hbm.at[cid, pl.ds(sid * STRIPE, STRIPE)])


@functools.partial(
    pl.kernel,
    mesh=_mesh,
    out_type=jax.ShapeDtypeStruct((N, C), jnp.bfloat16),
    compiler_params=_sc_params,
    scratch_types=[
        pltpu.VMEM((ECH,), jnp.int32),
        pltpu.VMEM((ECH,), jnp.int32),
        pltpu.VMEM((SB,), jnp.int32),
        pltpu.VMEM((SB,), jnp.int32),
        pltpu.VMEM((SB,), jnp.int32),
        pltpu.VMEM((SB,), jnp.int32),
        pltpu.VMEM((2, SB, CG), jnp.bfloat16),
        pltpu.VMEM_SHARED((N, CG), jnp.bfloat16),
        pltpu.SemaphoreType.DMA,
        pltpu.SemaphoreType.DMA,
    ],
)
def _scatter_kernel(hd_hbm, row_hbm, col_hbm, acc_hbm,
                    rowchunk, colchunk, rowidx0, colidx0, rowidx1, colidx1,
                    stage, spacc, sem_g, sem_s):
    cid = lax.axis_index("c")
    sid = lax.axis_index("s")
    rixs = (rowidx0, rowidx1)
    cixs = (colidx0, colidx1)

    for p in range(G // NSC):
        g = p * NSC + cid

        def _zz(j, carry):
            stage[0, j, :] = jnp.zeros((CG,), jnp.bfloat16)
            return carry

        lax.fori_loop(0, SB, _zz, 0)
        zh = [
            pltpu.async_copy(
                stage.at[0], spacc.at[pl.ds(sid * STRIPE + k * SB, SB)],
                sem_g)
            for k in range(STRIPE // SB)
        ]
        for h in zh:
            h.wait()
        plsc.subcore_barrier()

        def _chunk(cki, carry):
            e0 = sid * EPS + cki * ECH
            pltpu.sync_copy(row_hbm.at[pl.ds(e0, ECH)], rowchunk)
            pltpu.sync_copy(col_hbm.at[pl.ds(e0, ECH)], colchunk)

            gh, sh = {}, {}
            for k in range(ECH // SB):
                kb = k % 2
                if k >= 2:
                    sh[k - 2].wait()

                def _bld(j, carry2, _k=k, _kb=kb):
                    rv = rowchunk[pl.ds(_k * SB + j * 16, 16)]
                    cv = colchunk[pl.ds(_k * SB + j * 16, 16)]
                    rixs[_kb][pl.ds(j * 16, 16)] = rv * G + g
                    cixs[_kb][pl.ds(j * 16, 16)] = cv
                    return carry2

                lax.fori_loop(0, SB // 16, _bld, 0)
                gh[k] = pltpu.async_copy(
                    hd_hbm.at[rixs[kb]], stage.at[kb], sem_g)
                if k >= 1:
                    gh[k - 1].wait()
                    sh[k - 1] = pltpu.async_copy(
                        stage.at[(k - 1) % 2], spacc.at[cixs[(k - 1) % 2]],
                        sem_s, add=True)
            last = ECH // SB - 1
            gh[last].wait()
            sh[last] = pltpu.async_copy(
                stage.at[last % 2], spacc.at[cixs[last % 2]], sem_s,
                add=True)
            sh[last - 1].wait()
            sh[last].wait()
            return carry

        lax.fori_loop(0, EPS // ECH, _chunk, 0)
        plsc.subcore_barrier()

        dh = {}
        for k in range(STRIPE // SB):
            kb = k % 2
            if k >= 2:
                dh[k - 2].wait()
            r0 = sid * STRIPE + k * SB
            pltpu.sync_copy(spacc.at[pl.ds(r0, SB)], stage.at[kb])
            dh[k] = pltpu.async_copy(
                stage.at[kb],
                acc_hbm.at[pl.ds(r0, SB), pl.ds(g * CG, CG)], sem_s)
        dh[STRIPE // SB - 2].wait()
        dh[STRIPE // SB - 1].wait()


def _hd_body(x_ref, cnt_ref, w_ref, hd_ref, dis_ref):
    deg = cnt_ref[0:1, :] + cnt_ref[1:2, :] + 1.0
    dis_row = lax.rsqrt(deg)
    dis_col = dis_row.reshape(ROWBLK, 1)
    h = jnp.dot(x_ref[...], w_ref[...], preferred_element_type=jnp.float32)
    hd_ref[...] = (h * dis_col).astype(jnp.bfloat16)
    dis_ref[...] = dis_row.reshape(1, 1, ROWBLK)


_hd_call = pl.pallas_call(
    _hd_body,
    grid=(GRID,),
    in_specs=[
        pl.BlockSpec((ROWBLK, C), lambda i: (i, 0)),
        pl.BlockSpec((2, ROWBLK), lambda i: (0, i)),
        pl.BlockSpec((C, C), lambda i: (0, 0)),
    ],
    out_specs=[
        pl.BlockSpec((ROWBLK, C), lambda i: (i, 0)),
        pl.BlockSpec((1, 1, ROWBLK), lambda i: (i, 0, 0)),
    ],
    out_shape=[
        jax.ShapeDtypeStruct((N, C), jnp.bfloat16),
        jax.ShapeDtypeStruct((GRID, 1, ROWBLK), jnp.float32),
    ],
)


def _y_body(acc_ref, hd_ref, dis_ref, b_ref, y_ref, sums_ref):
    i = pl.program_id(0)
    acc = acc_ref[...].astype(jnp.float32)
    hd = hd_ref[...].astype(jnp.float32)
    dis_col = dis_ref[...].reshape(ROWBLK, 1)
    y = (acc + hd) * dis_col + b_ref[...]
    y_ref[...] = y
    s1 = jnp.sum(y, axis=0, keepdims=True)
    s2 = jnp.sum(y * y, axis=0, keepdims=True)
    blk = jnp.concatenate([s1, s2, jnp.zeros((6, C), jnp.float32)], axis=0)

    @pl.when(i == 0)
    def _():
        sums_ref[...] = blk

    @pl.when(i > 0)
    def _():
        sums_ref[...] = sums_ref[...] + blk


_y_call = pl.pallas_call(
    _y_body,
    grid=(GRID,),
    in_specs=[
        pl.BlockSpec((ROWBLK, C), lambda i: (i, 0)),
        pl.BlockSpec((ROWBLK, C), lambda i: (i, 0)),
        pl.BlockSpec((1, 1, ROWBLK), lambda i: (i, 0, 0)),
        pl.BlockSpec((1, C), lambda i: (0, 0)),
    ],
    out_specs=[
        pl.BlockSpec((ROWBLK, C), lambda i: (i, 0)),
        pl.BlockSpec((8, C), lambda i: (0, 0)),
    ],
    out_shape=[
        jax.ShapeDtypeStruct((N, C), jnp.float32),
        jax.ShapeDtypeStruct((8, C), jnp.float32),
    ],
)


def _bn_body(y_ref, a_ref, c_ref, o_ref):
    z = y_ref[...] * a_ref[...] + c_ref[...]
    o_ref[...] = jnp.where(z >= 0, z, 0.2 * z)


_bn_call = pl.pallas_call(
    _bn_body,
    grid=(GRID,),
    in_specs=[
        pl.BlockSpec((ROWBLK, C), lambda i: (i, 0)),
        pl.BlockSpec((1, C), lambda i: (0, 0)),
        pl.BlockSpec((1, C), lambda i: (0, 0)),
    ],
    out_specs=pl.BlockSpec((ROWBLK, C), lambda i: (i, 0)),
    out_shape=jax.ShapeDtypeStruct((N, C), jnp.float32),
)


def kernel(x, edge_index, batch_size, num_frames, num_joints, W, b, gamma, beta):
    row = edge_index[0]
    col = edge_index[1]

    cnt = _degree_kernel(col)

    hd, dis = _hd_call(x, cnt, W)

    acc = _scatter_kernel(hd.reshape(N * G, CG), row, col)

    y, sums = _y_call(acc, hd, dis, b.reshape(1, C))
    mean = sums[0] / N
    var = sums[1] / N - mean * mean
    inv = lax.rsqrt(var + 1e-5)
    a = inv * gamma
    c = beta - mean * a

    out = _bn_call(y, a.reshape(1, C), c.reshape(1, C))
    return out.reshape(64, 64, 25, C)


# speedup vs baseline: 1.1463x; 1.0197x over previous
import functools

import jax
import jax.numpy as jnp
from jax import lax
from jax.experimental import pallas as pl
from jax.experimental.pallas import tpu as pltpu
from jax.experimental.pallas import tpu_sc as plsc

N = 102400
E = 409600
C = 128
G = 4
CG = C // G
NSC = 2
NT = 16
EPT = E // (NSC * NT)
EPS = E // NT
ECH = 3200
SB = 400
STRIPE = N // NT
ROWBLK = 2048
GRID = N // ROWBLK

_mesh = plsc.VectorSubcoreMesh(core_axis_name="c", subcore_axis_name="s")
_sc_params = pltpu.CompilerParams(use_tc_tiling_on_sc=False)


@functools.partial(
    pl.kernel,
    mesh=_mesh,
    out_type=jax.ShapeDtypeStruct((NSC, N), jnp.float32),
    compiler_params=_sc_params,
    scratch_types=[
        pltpu.VMEM((EPT,), jnp.int32),
        pltpu.VMEM((1, 128), jnp.int32),
        pltpu.VMEM((128,), jnp.float32),
        pltpu.VMEM((STRIPE,), jnp.float32),
        pltpu.VMEM_SHARED((N,), jnp.float32),
    ],
)
def _degree_kernel(col_hbm, cnt_hbm, colchunk, colstage, ones_v, stage1d, spdeg):
    cid = lax.axis_index("c")
    sid = lax.axis_index("s")
    wid = sid * NSC + cid

    def _z(j, carry):
        stage1d[pl.ds(j * 16, 16)] = jnp.zeros((16,), jnp.float32)
        return carry

    lax.fori_loop(0, STRIPE // 16, _z, 0)
    for k in range(8):
        ones_v[pl.ds(k * 16, 16)] = jnp.ones((16,), jnp.float32)

    pltpu.sync_copy(stage1d, spdeg.at[pl.ds(sid * STRIPE, STRIPE)])
    plsc.subcore_barrier()

    pltpu.sync_copy(col_hbm.at[pl.ds(wid * EPT, EPT)], colchunk)

    def _hist(b, carry):
        for k in range(8):
            colstage[0, pl.ds(k * 16, 16)] = colchunk[pl.ds(b * 128 + k * 16, 16)]
        pltpu.sync_copy(ones_v, spdeg.at[colstage.at[0]], add=True)
        return carry

    lax.fori_loop(0, EPT // 128, _hist, 0)
    plsc.subcore_barrier()

    pltpu.sync_copy(spdeg.at[pl.ds(sid * STRIPE, STRIPE)], stage1d)
    pltpu.sync_copy(stage1d, cnt_hbm.at[cid, pl.ds(sid * STRIPE, STRIPE)])


@functools.partial(
    pl.kernel,
    mesh=_mesh,
    out_type=jax.ShapeDtypeStruct((N, C), jnp.bfloat16),
    compiler_params=_sc_params,
    scratch_types=[
        pltpu.VMEM((ECH,), jnp.int32),
        pltpu.VMEM((ECH,), jnp.int32),
        pltpu.VMEM((SB,), jnp.int32),
        pltpu.VMEM((SB,), jnp.int32),
        pltpu.VMEM((SB,), jnp.int32),
        pltpu.VMEM((SB,), jnp.int32),
        pltpu.VMEM((2, SB, CG), jnp.bfloat16),
        pltpu.VMEM_SHARED((N, CG), jnp.bfloat16),
        pltpu.SemaphoreType.DMA,
        pltpu.SemaphoreType.DMA,
    ],
)
def _scatter_kernel(hd_hbm, row_hbm, col_hbm, acc_hbm,
                    rowchunk, colchunk, rowidx0, colidx0, rowidx1, colidx1,
                    stage, spacc, sem_g, sem_s):
    cid = lax.axis_index("c")
    sid = lax.axis_index("s")
    rixs = (rowidx0, rowidx1)
    cixs = (colidx0, colidx1)

    for p in range(G // NSC):
        g = p * NSC + cid

        def _zz(j, carry):
            stage[0, j, :] = jnp.zeros((CG,), jnp.bfloat16)
            return carry

        lax.fori_loop(0, SB, _zz, 0)
        zh = [
            pltpu.async_copy(
                stage.at[0], spacc.at[pl.ds(sid * STRIPE + k * SB, SB)],
                sem_g)
            for k in range(STRIPE // SB)
        ]
        for h in zh:
            h.wait()
        plsc.subcore_barrier()

        def _chunk(cki, carry):
            e0 = sid * EPS + cki * ECH
            pltpu.sync_copy(row_hbm.at[pl.ds(e0, ECH)], rowchunk)
            pltpu.sync_copy(col_hbm.at[pl.ds(e0, ECH)], colchunk)

            gh, sh = {}, {}
            for k in range(ECH // SB):
                kb = k % 2
                if k >= 2:
                    sh[k - 2].wait()

                def _bld(j, carry2, _k=k, _kb=kb):
                    rv = rowchunk[pl.ds(_k * SB + j * 16, 16)]
                    cv = colchunk[pl.ds(_k * SB + j * 16, 16)]
                    rixs[_kb][pl.ds(j * 16, 16)] = rv * G + g
                    cixs[_kb][pl.ds(j * 16, 16)] = cv
                    return carry2

                lax.fori_loop(0, SB // 16, _bld, 0)
                gh[k] = pltpu.async_copy(
                    hd_hbm.at[rixs[kb]], stage.at[kb], sem_g)
                if k >= 1:
                    gh[k - 1].wait()
                    sh[k - 1] = pltpu.async_copy(
                        stage.at[(k - 1) % 2], spacc.at[cixs[(k - 1) % 2]],
                        sem_s, add=True)
            last = ECH // SB - 1
            gh[last].wait()
            sh[last] = pltpu.async_copy(
                stage.at[last % 2], spacc.at[cixs[last % 2]], sem_s,
                add=True)
            sh[last - 1].wait()
            sh[last].wait()
            return carry

        lax.fori_loop(0, EPS // ECH, _chunk, 0)
        plsc.subcore_barrier()

        dh = {}
        for k in range(STRIPE // SB):
            kb = k % 2
            if k >= 2:
                dh[k - 2].wait()
            r0 = sid * STRIPE + k * SB
            pltpu.sync_copy(spacc.at[pl.ds(r0, SB)], stage.at[kb])
            dh[k] = pltpu.async_copy(
                stage.at[kb],
                acc_hbm.at[pl.ds(r0, SB), pl.ds(g * CG, CG)], sem_s)
        dh[STRIPE // SB - 2].wait()
        dh[STRIPE // SB - 1].wait()


def _hd_body(x_ref, cnt_ref, w_ref, hd_ref, dis_ref):
    deg = cnt_ref[0:1, :] + cnt_ref[1:2, :] + 1.0
    dis_row = lax.rsqrt(deg)
    dis_col = dis_row.reshape(ROWBLK, 1)
    h = jnp.dot(x_ref[...], w_ref[...], preferred_element_type=jnp.float32)
    hd_ref[...] = (h * dis_col).astype(jnp.bfloat16)
    dis_ref[...] = dis_row.reshape(1, 1, ROWBLK)


_hd_call = pl.pallas_call(
    _hd_body,
    grid=(GRID,),
    in_specs=[
        pl.BlockSpec((ROWBLK, C), lambda i: (i, 0)),
        pl.BlockSpec((2, ROWBLK), lambda i: (0, i)),
        pl.BlockSpec((C, C), lambda i: (0, 0)),
    ],
    out_specs=[
        pl.BlockSpec((ROWBLK, C), lambda i: (i, 0)),
        pl.BlockSpec((1, 1, ROWBLK), lambda i: (i, 0, 0)),
    ],
    out_shape=[
        jax.ShapeDtypeStruct((N, C), jnp.bfloat16),
        jax.ShapeDtypeStruct((GRID, 1, ROWBLK), jnp.float32),
    ],
)


def _y_body(acc_ref, hd_ref, dis_ref, b_ref, y_ref, sums_ref):
    i = pl.program_id(0)
    acc = acc_ref[...].astype(jnp.float32)
    hd = hd_ref[...].astype(jnp.float32)
    dis_col = dis_ref[...].reshape(ROWBLK, 1)
    y = (acc + hd) * dis_col + b_ref[...]
    y_ref[...] = y.astype(jnp.bfloat16)
    s1 = jnp.sum(y, axis=0, keepdims=True)
    s2 = jnp.sum(y * y, axis=0, keepdims=True)
    blk = jnp.concatenate([s1, s2, jnp.zeros((6, C), jnp.float32)], axis=0)

    @pl.when(i == 0)
    def _():
        sums_ref[...] = blk

    @pl.when(i > 0)
    def _():
        sums_ref[...] = sums_ref[...] + blk


_y_call = pl.pallas_call(
    _y_body,
    grid=(GRID,),
    in_specs=[
        pl.BlockSpec((ROWBLK, C), lambda i: (i, 0)),
        pl.BlockSpec((ROWBLK, C), lambda i: (i, 0)),
        pl.BlockSpec((1, 1, ROWBLK), lambda i: (i, 0, 0)),
        pl.BlockSpec((1, C), lambda i: (0, 0)),
    ],
    out_specs=[
        pl.BlockSpec((ROWBLK, C), lambda i: (i, 0)),
        pl.BlockSpec((8, C), lambda i: (0, 0)),
    ],
    out_shape=[
        jax.ShapeDtypeStruct((N, C), jnp.bfloat16),
        jax.ShapeDtypeStruct((8, C), jnp.float32),
    ],
)


def _bn_body(y_ref, a_ref, c_ref, o_ref):
    z = y_ref[...].astype(jnp.float32) * a_ref[...] + c_ref[...]
    o_ref[...] = jnp.where(z >= 0, z, 0.2 * z)


_bn_call = pl.pallas_call(
    _bn_body,
    grid=(GRID,),
    in_specs=[
        pl.BlockSpec((ROWBLK, C), lambda i: (i, 0)),
        pl.BlockSpec((1, C), lambda i: (0, 0)),
        pl.BlockSpec((1, C), lambda i: (0, 0)),
    ],
    out_specs=pl.BlockSpec((ROWBLK, C), lambda i: (i, 0)),
    out_shape=jax.ShapeDtypeStruct((N, C), jnp.float32),
)


def kernel(x, edge_index, batch_size, num_frames, num_joints, W, b, gamma, beta):
    row = edge_index[0]
    col = edge_index[1]

    cnt = _degree_kernel(col)

    hd, dis = _hd_call(x, cnt, W)

    acc = _scatter_kernel(hd.reshape(N * G, CG), row, col)

    y, sums = _y_call(acc, hd, dis, b.reshape(1, C))
    mean = sums[0] / N
    var = sums[1] / N - mean * mean
    inv = lax.rsqrt(var + 1e-5)
    a = inv * gamma
    c = beta - mean * a

    out = _bn_call(y, a.reshape(1, C), c.reshape(1, C))
    return out.reshape(64, 64, 25, C)
